# baseline (speedup 1.0000x reference)
import jax
import jax.numpy as jnp
from jax import lax
from jax.experimental import pallas as pl
from jax.experimental.pallas import tpu as pltpu
from jax.experimental.pallas import tpu_sc as plsc

VOCAB = 1000000
EMBED_DIM = 64
PACK_DIM = 128
PACK_ROWS = VOCAB // 2
BATCH = 4096
SEQ = 20

_INFO = plsc.get_sparse_core_info()
_NC, _NS = _INFO.num_cores, _INFO.num_subcores
_NW = _NC * _NS
_ROWS = BATCH * SEQ
_CHUNK = 128
_PER_W = _ROWS // _NW
_NCHUNK = _PER_W // _CHUNK
_NBUF = 5
_NROUND = _NCHUNK // _NBUF


def _gather_body(table_hbm, idx_hbm, out_hbm, idx_v, rows_v, gsem):
    wid = lax.axis_index("s") * _NC + lax.axis_index("c")
    base = wid * _PER_W
    pltpu.sync_copy(idx_hbm.at[wid], idx_v)

    def round_(r, carry):
        copies = []
        for b in range(_NBUF):
            copies.append(
                pltpu.async_copy(
                    table_hbm.at[idx_v.at[r * _NBUF + b]],
                    rows_v.at[pl.ds(b * _CHUNK, _CHUNK)],
                    gsem,
                )
            )
        for c in copies:
            c.wait()
        pltpu.sync_copy(
            rows_v,
            out_hbm.at[pl.ds(base + r * _NBUF * _CHUNK, _NBUF * _CHUNK)],
        )
        return carry

    lax.fori_loop(0, _NROUND, round_, 0)


@jax.jit
def _embed_gather(xhalf, packed):
    mesh = plsc.VectorSubcoreMesh(core_axis_name="c", subcore_axis_name="s")
    k = pl.kernel(
        _gather_body,
        out_type=jax.ShapeDtypeStruct((_ROWS, PACK_DIM), jnp.float32),
        mesh=mesh,
        scratch_types=[
            pltpu.VMEM((_NCHUNK, _CHUNK), jnp.int32),
            pltpu.VMEM((_NBUF * _CHUNK, PACK_DIM), jnp.float32),
            pltpu.SemaphoreType.DMA,
        ],
        compiler_params=pltpu.CompilerParams(use_tc_tiling_on_sc=True),
    )
    return k(packed, xhalf.reshape(_NW, _NCHUNK, _CHUNK))


def kernel(x, input_embedding):
    packed = jnp.concatenate(
        [input_embedding[0::2], input_embedding[1::2]], axis=1
    )
    xf = x.reshape(-1)
    pairs = _embed_gather(xf // 2, packed)
    out = jnp.where(
        (xf % 2 == 0)[:, None], pairs[:, :EMBED_DIM], pairs[:, EMBED_DIM:]
    )
    return out.reshape(BATCH, SEQ, EMBED_DIM)


# speedup vs baseline: 13.9542x; 13.9542x over previous
import jax
import jax.numpy as jnp
from jax import lax
from jax.experimental import pallas as pl
from jax.experimental.pallas import tpu as pltpu
from jax.experimental.pallas import tpu_sc as plsc

VOCAB = 1000000
EMBED_DIM = 64
PAD_DIM = 128
BATCH = 4096
SEQ = 20

_INFO = plsc.get_sparse_core_info()
_NC, _NS = _INFO.num_cores, _INFO.num_subcores
_NW = _NC * _NS
_ROWS = BATCH * SEQ
_CHUNK = 128
_PER_W = _ROWS // _NW
_NCHUNK = _PER_W // _CHUNK
_NBUF = 5
_NROUND = _NCHUNK // _NBUF


def _gather_body(table_hbm, idx_hbm, out_hbm, idx_v, rows_v, gsem):
    wid = lax.axis_index("s") * _NC + lax.axis_index("c")
    base = wid * _PER_W
    pltpu.sync_copy(idx_hbm.at[wid], idx_v)

    def round_(r, carry):
        copies = []
        for b in range(_NBUF):
            copies.append(
                pltpu.async_copy(
                    table_hbm.at[idx_v.at[r * _NBUF + b]],
                    rows_v.at[pl.ds(b * _CHUNK, _CHUNK)],
                    gsem,
                )
            )
        for c in copies:
            c.wait()
        pltpu.sync_copy(
            rows_v,
            out_hbm.at[pl.ds(base + r * _NBUF * _CHUNK, _NBUF * _CHUNK)],
        )
        return carry

    lax.fori_loop(0, _NROUND, round_, 0)


@jax.jit
def _embed_gather(x, table):
    mesh = plsc.VectorSubcoreMesh(core_axis_name="c", subcore_axis_name="s")
    k = pl.kernel(
        _gather_body,
        out_type=jax.ShapeDtypeStruct((_ROWS, PAD_DIM), jnp.float32),
        mesh=mesh,
        scratch_types=[
            pltpu.VMEM((_NCHUNK, _CHUNK), jnp.int32),
            pltpu.VMEM((_NBUF * _CHUNK, PAD_DIM), jnp.float32),
            pltpu.SemaphoreType.DMA,
        ],
        compiler_params=pltpu.CompilerParams(use_tc_tiling_on_sc=True),
    )
    return k(table, x.reshape(_NW, _NCHUNK, _CHUNK))


def kernel(x, input_embedding):
    padded = jnp.pad(input_embedding, ((0, 0), (0, PAD_DIM - EMBED_DIM)))
    out = _embed_gather(x, padded)
    return out[:, :EMBED_DIM].reshape(BATCH, SEQ, EMBED_DIM)
